# baseline (speedup 1.0000x reference)
import functools

import jax
import jax.numpy as jnp
from jax import lax
from jax.experimental import pallas as pl
from jax.experimental.pallas import tpu as pltpu
from jax.experimental.pallas import tpu_sc as plsc

D = 64
CB = 16


def _make_encoder(batch, seq):
    info = plsc.get_sparse_core_info()
    nc, ns = info.num_cores, info.num_subcores
    nw = nc * ns
    b_per_w = batch // nw
    assert batch % nw == 0 and b_per_w % CB == 0
    n_chunks = b_per_w // CB

    mesh = plsc.VectorSubcoreMesh(core_axis_name="c", subcore_axis_name="s")

    @functools.partial(
        pl.kernel,
        mesh=mesh,
        compiler_params=pltpu.CompilerParams(use_tc_tiling_on_sc=False),
        out_type=jax.ShapeDtypeStruct((batch, seq, D), jnp.float32),
        scratch_types=[
            pltpu.VMEM((CB, seq), jnp.int32),
            pltpu.VMEM((CB, seq), jnp.int32),
            pltpu.VMEM((CB, seq, D), jnp.float32),
            pltpu.SemaphoreType.DMA,
            pltpu.SemaphoreType.DMA,
        ],
    )
    def enc(node_hbm, deg_hbm, nidx_hbm, didx_hbm, out_hbm,
            nidx_v, didx_v, rows_v, nsem, dsem):
        wid = lax.axis_index("s") * nc + lax.axis_index("c")
        base = wid * b_per_w

        def chunk_body(ci, carry):
            b0 = base + ci * CB
            pltpu.sync_copy(nidx_hbm.at[pl.ds(b0, CB)], nidx_v)
            pltpu.sync_copy(didx_hbm.at[pl.ds(b0, CB)], didx_v)
            copies = []
            for j in range(CB):
                copies.append(pltpu.async_copy(
                    node_hbm.at[nidx_v.at[j]], rows_v.at[j], nsem))
            for cp in copies:
                cp.wait()
            copies = []
            for j in range(CB):
                copies.append(pltpu.async_copy(
                    deg_hbm.at[didx_v.at[j]], rows_v.at[j], dsem, add=True))
            for cp in copies:
                cp.wait()
            pltpu.sync_copy(rows_v, out_hbm.at[pl.ds(b0, CB)])
            return carry

        lax.fori_loop(0, n_chunks, chunk_body, 0)

    return enc


@jax.jit
def kernel(user_seq, user_degree, node_table, degree_table):
    b, s = user_seq.shape
    enc = _make_encoder(b, s)
    return enc(node_table, degree_table, user_seq, user_degree)
